# baseline (speedup 1.0000x reference)
import functools

import jax
import jax.numpy as jnp
from jax import lax
from jax.experimental import pallas as pl
from jax.experimental.pallas import tpu as pltpu
from jax.experimental.pallas import tpu_sc as plsc

N = 10000
E = 320000
D = 128

NC = 2
NS = 16
NW = NC * NS
C = 128

EPW = -(-E // NW)
NCHUNK = -(-EPW // C)
E_PAD = NW * NCHUNK * C

NPAD = 10112
RPS = NPAD // NS


def _sc_agg_body(h_hbm, src_hbm, dst_hbm, zeros_hbm, out_hbm,
                 src_v, dst_v, rows_v, agg_sh, sem):
    c = lax.axis_index("c")
    s = lax.axis_index("s")
    wid = c * NS + s

    pltpu.sync_copy(zeros_hbm.at[pl.ds(s * RPS, RPS)],
                    agg_sh.at[pl.ds(s * RPS, RPS)])
    pltpu.sync_copy(src_hbm.at[wid], src_v)
    pltpu.sync_copy(dst_hbm.at[wid], dst_v)
    plsc.subcore_barrier()

    def chunk(j, carry):
        pltpu.async_copy(h_hbm.at[src_v.at[j]], rows_v, sem).wait()
        pltpu.sync_copy(rows_v, agg_sh.at[dst_v.at[j]], add=True)
        return carry

    lax.fori_loop(0, NCHUNK, chunk, 0)
    plsc.subcore_barrier()

    pltpu.sync_copy(agg_sh.at[pl.ds(s * RPS, RPS)],
                    out_hbm.at[c, pl.ds(s * RPS, RPS)])


@jax.jit
def _sc_agg(h, src_w, dst_w, zeros):
    mesh = plsc.VectorSubcoreMesh(core_axis_name="c", subcore_axis_name="s",
                                  num_cores=NC, num_subcores=NS)
    return pl.kernel(
        _sc_agg_body,
        out_type=jax.ShapeDtypeStruct((NC, NPAD, D), jnp.float32),
        mesh=mesh,
        scratch_types=[
            pltpu.VMEM((NCHUNK, C), jnp.int32),
            pltpu.VMEM((NCHUNK, C), jnp.int32),
            pltpu.VMEM((C, D), jnp.float32),
            pltpu.VMEM_SHARED((NPAD, D), jnp.float32),
            pltpu.SemaphoreType.DMA,
        ],
    )(h, src_w, dst_w, zeros)


def _tc_mlp_body(h_ref, p_ref, w0, b0, w1, b1, w2, b2, out_ref):
    t = h_ref[...] + p_ref[0] + p_ref[1]
    t = jnp.maximum(jnp.dot(t, w0[...], preferred_element_type=jnp.float32)
                    + b0[...], 0.0)
    t = jnp.maximum(jnp.dot(t, w1[...], preferred_element_type=jnp.float32)
                    + b1[...], 0.0)
    out_ref[...] = (jnp.dot(t, w2[...], preferred_element_type=jnp.float32)
                    + b2[...])


def _tc_mlp(h, parts, Ws, bs):
    BN = 1000
    grid = N // BN
    d0, d1 = Ws[0].shape[1], Ws[1].shape[1]
    return pl.pallas_call(
        _tc_mlp_body,
        grid=(grid,),
        in_specs=[
            pl.BlockSpec((BN, D), lambda i: (i, 0)),
            pl.BlockSpec((NC, BN, D), lambda i: (0, i, 0)),
            pl.BlockSpec((D, d0), lambda i: (0, 0)),
            pl.BlockSpec((1, d0), lambda i: (0, 0)),
            pl.BlockSpec((d0, d1), lambda i: (0, 0)),
            pl.BlockSpec((1, d1), lambda i: (0, 0)),
            pl.BlockSpec((d1, D), lambda i: (0, 0)),
            pl.BlockSpec((1, D), lambda i: (0, 0)),
        ],
        out_specs=pl.BlockSpec((BN, D), lambda i: (i, 0)),
        out_shape=jax.ShapeDtypeStruct((N, D), jnp.float32),
    )(h, parts, Ws[0], bs[0].reshape(1, -1), Ws[1], bs[1].reshape(1, -1),
      Ws[2], bs[2].reshape(1, -1))


def kernel(x, edge_index, params):
    src = edge_index[0].astype(jnp.int32)
    dst = edge_index[1].astype(jnp.int32)
    pad = E_PAD - E
    src_w = jnp.concatenate([src, jnp.zeros((pad,), jnp.int32)])
    dst_w = jnp.concatenate([dst, jnp.full((pad,), N, jnp.int32)])
    src_w = src_w.reshape(NW, NCHUNK, C)
    dst_w = dst_w.reshape(NW, NCHUNK, C)
    zeros = jnp.zeros((NPAD, D), jnp.float32)

    h = x
    for (Ws, bs) in params:
        parts = _sc_agg(h, src_w, dst_w, zeros)
        h = _tc_mlp(h, parts, Ws, bs)
    return h
